# Initial kernel scaffold; baseline (speedup 1.0000x reference)
import jax
import jax.numpy as jnp
from jax.experimental import pallas as pl


def kernel(z, embedding):
    raise NotImplementedError("write your pallas kernel here")


# speedup vs baseline: 1.5605x; 1.5605x over previous
import jax
import jax.numpy as jnp
from jax.experimental import pallas as pl
from jax.experimental.pallas import tpu as pltpu

_CODEBOOK = 1024
_D = 64
_COMMIT = 0.25


def _vq_body(z_ref, emb_ref, zq_ref, idx_ref, loss_ref):
    b = pl.program_id(0)
    nb = pl.num_programs(0)
    zb = z_ref[0]
    emb = emb_ref[...]

    zsq = jnp.sum(zb * zb, axis=0)
    esq = jnp.sum(emb * emb, axis=1)
    scores = jax.lax.dot_general(
        zb, emb, (((0,), (1,)), ((), ())),
        preferred_element_type=jnp.float32)
    dist = (zsq[:, None] - 2.0 * scores) + esq[None, :]

    m = jnp.min(dist, axis=1, keepdims=True)
    c_iota = jax.lax.broadcasted_iota(jnp.int32, dist.shape, 1)
    idx = jnp.min(jnp.where(dist == m, c_iota, _CODEBOOK), axis=1)
    idx_ref[0, 0, :] = idx

    onehot = (c_iota == idx[:, None]).astype(jnp.float32)
    zq = jax.lax.dot_general(
        emb, onehot, (((0,), (1,)), ((), ())),
        preferred_element_type=jnp.float32)
    zq_ref[0] = zq

    diff = zb - zq
    part = jnp.sum(diff * diff)

    @pl.when(b == 0)
    def _init():
        loss_ref[0, 0] = jnp.float32(0.0)

    loss_ref[0, 0] += part


def kernel(z, embedding):
    B, D, H, W = z.shape
    P = H * W
    z3 = z.reshape(B, D, P)

    zq3, idx3, loss_raw = pl.pallas_call(
        _vq_body,
        grid=(B,),
        in_specs=[
            pl.BlockSpec((1, D, P), lambda b: (b, 0, 0)),
            pl.BlockSpec((_CODEBOOK, D), lambda b: (0, 0)),
        ],
        out_specs=[
            pl.BlockSpec((1, D, P), lambda b: (b, 0, 0)),
            pl.BlockSpec((1, 1, P), lambda b: (b, 0, 0)),
            pl.BlockSpec((1, 1), lambda b: (0, 0),
                         memory_space=pltpu.SMEM),
        ],
        out_shape=[
            jax.ShapeDtypeStruct((B, D, P), jnp.float32),
            jax.ShapeDtypeStruct((B, 1, P), jnp.int32),
            jax.ShapeDtypeStruct((1, 1), jnp.float32),
        ],
    )(z3, embedding)

    z_q = zq3.reshape(B, D, H, W)
    indices = idx3.reshape(B, H, W)
    loss = loss_raw[0, 0] * (_COMMIT / (B * P * D))
    return (z_q, loss, indices)
